# baseline (speedup 1.0000x reference)
import functools

import jax
import jax.numpy as jnp
from jax.experimental import pallas as pl
from jax.experimental.pallas import tpu as pltpu

N = 10000
F = 512
BM = 400
BH = BM // 2


def _xw_kernel(x_ref, w_ref, out_ref):
    out_ref[...] = jnp.dot(
        x_ref[...].astype(jnp.bfloat16),
        w_ref[...],
        preferred_element_type=jnp.float32,
    ).astype(jnp.bfloat16)


def _layer1_kernel(adja_ref, adjb_ref, s_ref, w2_ref, b1_ref, out_ref):
    s = s_ref[...]
    acc_a = jnp.dot(
        adja_ref[...].astype(jnp.bfloat16), s, preferred_element_type=jnp.float32
    )
    acc_b = jnp.dot(
        adjb_ref[...].astype(jnp.bfloat16), s, preferred_element_type=jnp.float32
    )
    h_a = jnp.maximum(acc_a + b1_ref[...], 0.0)
    h_b = jnp.maximum(acc_b + b1_ref[...], 0.0)
    w2 = w2_ref[...]
    out_ref[:BH, :] = jnp.dot(
        h_a.astype(jnp.bfloat16), w2, preferred_element_type=jnp.float32
    ).astype(jnp.bfloat16)
    out_ref[BH:, :] = jnp.dot(
        h_b.astype(jnp.bfloat16), w2, preferred_element_type=jnp.float32
    ).astype(jnp.bfloat16)


def _layer2_kernel(adja_ref, adjb_ref, hw_ref, b2_ref, out_ref):
    hwv = hw_ref[...]
    b2 = b2_ref[...]
    out_ref[:BH, :] = (
        jnp.dot(
            adja_ref[...].astype(jnp.bfloat16),
            hwv,
            preferred_element_type=jnp.float32,
        )
        + b2
    )
    out_ref[BH:, :] = (
        jnp.dot(
            adjb_ref[...].astype(jnp.bfloat16),
            hwv,
            preferred_element_type=jnp.float32,
        )
        + b2
    )


@jax.jit
def kernel(x, adj, W1, b1, W2, b2):
    grid = (N // BM,)
    params = pltpu.CompilerParams(dimension_semantics=("parallel",))

    s1 = pl.pallas_call(
        _xw_kernel,
        grid=grid,
        in_specs=[
            pl.BlockSpec((BM, F), lambda i: (i, 0)),
            pl.BlockSpec((F, F), lambda i: (0, 0)),
        ],
        out_specs=pl.BlockSpec((BM, F), lambda i: (i, 0)),
        out_shape=jax.ShapeDtypeStruct((N, F), jnp.bfloat16),
        compiler_params=params,
    )(x, W1.astype(jnp.bfloat16))

    hw = pl.pallas_call(
        _layer1_kernel,
        grid=grid,
        in_specs=[
            pl.BlockSpec((BH, N), lambda i: (2 * i, 0)),
            pl.BlockSpec((BH, N), lambda i: (2 * i + 1, 0)),
            pl.BlockSpec((N, F), lambda i: (0, 0)),
            pl.BlockSpec((F, F), lambda i: (0, 0)),
            pl.BlockSpec((1, F), lambda i: (0, 0)),
        ],
        out_specs=pl.BlockSpec((BM, F), lambda i: (i, 0)),
        out_shape=jax.ShapeDtypeStruct((N, F), jnp.bfloat16),
        compiler_params=params,
    )(adj, adj, s1, W2.astype(jnp.bfloat16), b1.reshape(1, F))

    out = pl.pallas_call(
        _layer2_kernel,
        grid=grid,
        in_specs=[
            pl.BlockSpec((BH, N), lambda i: (2 * i, 0)),
            pl.BlockSpec((BH, N), lambda i: (2 * i + 1, 0)),
            pl.BlockSpec((N, F), lambda i: (0, 0)),
            pl.BlockSpec((1, F), lambda i: (0, 0)),
        ],
        out_specs=pl.BlockSpec((BM, F), lambda i: (i, 0)),
        out_shape=jax.ShapeDtypeStruct((N, F), jnp.float32),
        compiler_params=params,
    )(adj, adj, hw, b2.reshape(1, F))

    return out


# speedup vs baseline: 1.0190x; 1.0190x over previous
import functools

import jax
import jax.numpy as jnp
from jax.experimental import pallas as pl
from jax.experimental.pallas import tpu as pltpu

N = 10000
F = 512
BM = 400


def _xw_kernel(x_ref, w_ref, out_ref):
    out_ref[...] = jnp.dot(
        x_ref[...].astype(jnp.bfloat16),
        w_ref[...],
        preferred_element_type=jnp.float32,
    ).astype(jnp.bfloat16)


def _layer1_kernel(adj_ref, s_ref, w2_ref, b1_ref, out_ref):
    acc = jnp.dot(
        adj_ref[...].astype(jnp.bfloat16),
        s_ref[...],
        preferred_element_type=jnp.float32,
    )
    h = jnp.maximum(acc + b1_ref[...], 0.0)
    out_ref[...] = jnp.dot(
        h.astype(jnp.bfloat16),
        w2_ref[...],
        preferred_element_type=jnp.float32,
    ).astype(jnp.bfloat16)


def _layer2_kernel(adj_ref, hw_ref, b2_ref, out_ref):
    out_ref[...] = (
        jnp.dot(
            adj_ref[...].astype(jnp.bfloat16),
            hw_ref[...],
            preferred_element_type=jnp.float32,
        )
        + b2_ref[...]
    )


@jax.jit
def kernel(x, adj, W1, b1, W2, b2):
    grid = (N // BM,)
    params = pltpu.CompilerParams(dimension_semantics=("parallel",))

    s1 = pl.pallas_call(
        _xw_kernel,
        grid=grid,
        in_specs=[
            pl.BlockSpec((BM, F), lambda i: (i, 0)),
            pl.BlockSpec((F, F), lambda i: (0, 0)),
        ],
        out_specs=pl.BlockSpec((BM, F), lambda i: (i, 0)),
        out_shape=jax.ShapeDtypeStruct((N, F), jnp.bfloat16),
        compiler_params=params,
    )(x, W1.astype(jnp.bfloat16))

    hw = pl.pallas_call(
        _layer1_kernel,
        grid=grid,
        in_specs=[
            pl.BlockSpec((BM, N), lambda i: (i, 0)),
            pl.BlockSpec((N, F), lambda i: (0, 0)),
            pl.BlockSpec((F, F), lambda i: (0, 0)),
            pl.BlockSpec((1, F), lambda i: (0, 0)),
        ],
        out_specs=pl.BlockSpec((BM, F), lambda i: (i, 0)),
        out_shape=jax.ShapeDtypeStruct((N, F), jnp.bfloat16),
        compiler_params=params,
    )(adj, s1, W2.astype(jnp.bfloat16), b1.reshape(1, F))

    out = pl.pallas_call(
        _layer2_kernel,
        grid=grid,
        in_specs=[
            pl.BlockSpec((BM, N), lambda i: (i, 0)),
            pl.BlockSpec((N, F), lambda i: (0, 0)),
            pl.BlockSpec((1, F), lambda i: (0, 0)),
        ],
        out_specs=pl.BlockSpec((BM, F), lambda i: (i, 0)),
        out_shape=jax.ShapeDtypeStruct((N, F), jnp.float32),
        compiler_params=params,
    )(adj, hw, b2.reshape(1, F))

    return out
